# Initial kernel scaffold; baseline (speedup 1.0000x reference)
import jax
import jax.numpy as jnp
from jax.experimental import pallas as pl


def kernel(x, weight):
    raise NotImplementedError("write your pallas kernel here")


# speedup vs baseline: 2.9587x; 2.9587x over previous
import functools

import jax
import jax.numpy as jnp
from jax import lax
from jax.experimental import pallas as pl
from jax.experimental.pallas import tpu as pltpu
from jax.experimental.pallas import tpu_sc as plsc

_INFO = plsc.get_sparse_core_info()
_NC = _INFO.num_cores
_NS = _INFO.num_subcores
_NW = _NC * _NS

_CHUNK = 128


def _make_gather(n_idx: int, hidden: int, vocab: int):
    assert n_idx % (_NW * _CHUNK) == 0
    per_w = n_idx // _NW
    n_chunks = per_w // _CHUNK

    mesh = plsc.VectorSubcoreMesh(core_axis_name="c", subcore_axis_name="s")

    @functools.partial(
        pl.kernel,
        out_type=jax.ShapeDtypeStruct((n_idx, hidden), jnp.float32),
        mesh=mesh,
        scratch_types=[
            pltpu.VMEM((n_chunks, _CHUNK), jnp.int32),
            pltpu.VMEM((_CHUNK, hidden), jnp.float32),
            pltpu.SemaphoreType.DMA,
        ],
    )
    def gather_kernel(table_hbm, idx_hbm, out_hbm, idx_v, rows_v, sem):
        wid = lax.axis_index("s") * _NC + lax.axis_index("c")
        row_base = wid * n_chunks
        pltpu.sync_copy(idx_hbm.at[wid], idx_v)

        def body(j, carry):
            pltpu.async_copy(table_hbm.at[idx_v.at[j]], rows_v, sem).wait()
            pltpu.sync_copy(
                rows_v, out_hbm.at[pl.ds((row_base + j) * _CHUNK, _CHUNK)]
            )
            return carry

        lax.fori_loop(0, n_chunks, body, 0)

    return gather_kernel


@jax.jit
def kernel(x, weight):
    b, s = x.shape
    vocab, hidden = weight.shape
    n_idx = b * s
    idx3d = x.reshape(_NW, n_idx // (_NW * _CHUNK), _CHUNK).astype(jnp.int32)
    out = _make_gather(n_idx, hidden, vocab)(weight, idx3d)
    return out.reshape(b, s, hidden)


# speedup vs baseline: 3.1131x; 1.0522x over previous
import functools

import jax
import jax.numpy as jnp
from jax import lax
from jax.experimental import pallas as pl
from jax.experimental.pallas import tpu as pltpu
from jax.experimental.pallas import tpu_sc as plsc

_INFO = plsc.get_sparse_core_info()
_NC = _INFO.num_cores
_NS = _INFO.num_subcores
_NW = _NC * _NS

_CHUNK = 128


def _make_gather(n_idx: int, hidden: int, vocab: int):
    assert n_idx % (_NW * _CHUNK) == 0
    per_w = n_idx // _NW
    n_chunks = per_w // _CHUNK
    assert n_chunks >= 2 and n_chunks % 2 == 0

    mesh = plsc.VectorSubcoreMesh(core_axis_name="c", subcore_axis_name="s")

    @functools.partial(
        pl.kernel,
        out_type=jax.ShapeDtypeStruct((n_idx, hidden), jnp.float32),
        mesh=mesh,
        scratch_types=[
            pltpu.VMEM((n_chunks, _CHUNK), jnp.int32),
            pltpu.VMEM((2, _CHUNK, hidden), jnp.float32),
            pltpu.SemaphoreType.DMA,
            pltpu.SemaphoreType.DMA,
            pltpu.SemaphoreType.DMA,
            pltpu.SemaphoreType.DMA,
        ],
    )
    def gather_kernel(table_hbm, idx_hbm, out_hbm, idx_v, rows_v, g0, g1, w0, w1):
        gsem = (g0, g1)
        wsem = (w0, w1)
        wid = lax.axis_index("s") * _NC + lax.axis_index("c")
        row_base = wid * n_chunks
        pltpu.sync_copy(idx_hbm.at[wid], idx_v)

        def start_gather(i, p):
            pltpu.async_copy(table_hbm.at[idx_v.at[i]], rows_v.at[p], gsem[p])

        def wait_gather(p):
            pltpu.make_async_copy(
                table_hbm.at[idx_v.at[0]], rows_v.at[p], gsem[p]
            ).wait()

        def start_wb(i, p):
            pltpu.async_copy(
                rows_v.at[p],
                out_hbm.at[pl.ds((row_base + i) * _CHUNK, _CHUNK)],
                wsem[p],
            )

        def wait_wb(p):
            pltpu.make_async_copy(
                rows_v.at[p], out_hbm.at[pl.ds(0, _CHUNK)], wsem[p]
            ).wait()

        start_gather(0, 0)
        wait_gather(0)
        start_wb(0, 0)
        start_gather(1, 1)

        @pl.loop(1, n_chunks - 1, step=2)
        def _body(j):
            for b in range(2):
                i = j + b
                p = (1 + b) % 2
                pn = 1 - p
                wait_gather(p)
                start_wb(i, p)
                wait_wb(pn)
                start_gather(i + 1, pn)

        wait_gather(1)
        start_wb(n_chunks - 1, 1)
        wait_wb(0)
        wait_wb(1)

    return gather_kernel


@jax.jit
def kernel(x, weight):
    b, s = x.shape
    vocab, hidden = weight.shape
    n_idx = b * s
    idx3d = x.reshape(_NW, n_idx // (_NW * _CHUNK), _CHUNK).astype(jnp.int32)
    out = _make_gather(n_idx, hidden, vocab)(weight, idx3d)
    return out.reshape(b, s, hidden)


# speedup vs baseline: 4.1707x; 1.3398x over previous
import functools

import jax
import jax.numpy as jnp
from jax import lax
from jax.experimental import pallas as pl
from jax.experimental.pallas import tpu as pltpu
from jax.experimental.pallas import tpu_sc as plsc

_INFO = plsc.get_sparse_core_info()
_NC = _INFO.num_cores
_NS = _INFO.num_subcores
_NW = _NC * _NS


def _make_gather(n_rows: int, seq: int, hidden: int, vocab: int):
    assert n_rows % _NW == 0
    rows_per_w = n_rows // _NW
    assert rows_per_w >= 2 and rows_per_w % 2 == 0

    mesh = plsc.VectorSubcoreMesh(core_axis_name="c", subcore_axis_name="s")

    @functools.partial(
        pl.kernel,
        out_type=jax.ShapeDtypeStruct((n_rows, seq, hidden), jnp.float32),
        mesh=mesh,
        scratch_types=[
            pltpu.VMEM((rows_per_w, seq), jnp.int32),
            pltpu.VMEM((2, seq, hidden), jnp.float32),
            pltpu.SemaphoreType.DMA,
            pltpu.SemaphoreType.DMA,
            pltpu.SemaphoreType.DMA,
            pltpu.SemaphoreType.DMA,
        ],
    )
    def gather_kernel(table_hbm, idx_hbm, out_hbm, idx_v, rows_v, g0, g1, w0, w1):
        gsem = (g0, g1)
        wsem = (w0, w1)
        wid = lax.axis_index("s") * _NC + lax.axis_index("c")
        row_base = wid * rows_per_w
        pltpu.sync_copy(idx_hbm.at[pl.ds(row_base, rows_per_w)], idx_v)

        def start_gather(i, p):
            pltpu.async_copy(table_hbm.at[idx_v.at[i]], rows_v.at[p], gsem[p])

        def wait_gather(p):
            pltpu.make_async_copy(
                table_hbm.at[idx_v.at[0]], rows_v.at[p], gsem[p]
            ).wait()

        def start_wb(i, p):
            pltpu.async_copy(rows_v.at[p], out_hbm.at[row_base + i], wsem[p])

        def wait_wb(p):
            pltpu.make_async_copy(rows_v.at[p], out_hbm.at[0], wsem[p]).wait()

        start_gather(0, 0)
        wait_gather(0)
        start_wb(0, 0)
        start_gather(1, 1)

        @pl.loop(1, rows_per_w - 1, step=2)
        def _body(j):
            for b in range(2):
                i = j + b
                p = (1 + b) % 2
                pn = 1 - p
                wait_gather(p)
                start_wb(i, p)
                wait_wb(pn)
                start_gather(i + 1, pn)

        wait_gather(1)
        start_wb(rows_per_w - 1, 1)
        wait_wb(0)
        wait_wb(1)

    return gather_kernel


@jax.jit
def kernel(x, weight):
    b, s = x.shape
    vocab, hidden = weight.shape
    return _make_gather(b, s, hidden, vocab)(weight, x.astype(jnp.int32))
